# baseline (speedup 1.0000x reference)
import functools

import jax
import jax.numpy as jnp
from jax import lax
from jax.experimental import pallas as pl
from jax.experimental.pallas import tpu as pltpu

TILE = 128
HEADS = 8
CH = 64
HC = HEADS * CH
OUT = 128
LOG2E = 1.4426950408889634


def _full_spec(shape):
    return pl.BlockSpec(shape, lambda *_: (0,) * len(shape))


def _gat_relation(xd_b, xs_b, as_ref, ad_ref, mask):
    e_src = lax.dot_general(as_ref[...], xs_b, (((1,), (1,)), ((), ())),
                            preferred_element_type=jnp.float32)
    e_dst = jnp.dot(xd_b, ad_ref[...],
                    preferred_element_type=jnp.float32)
    outs = []
    for h in range(HEADS):
        s = e_dst[:, h:h + 1] + e_src[h:h + 1, :]
        s = jnp.maximum(s, 0.2 * s)
        p = jnp.exp2(s) * mask
        denom = jnp.sum(p, axis=1, keepdims=True)
        inv = pl.reciprocal(jnp.where(denom > 0, denom, 1.0), approx=True)
        outs.append(jnp.dot(p, xs_b[:, h * CH:(h + 1) * CH],
                            preferred_element_type=jnp.float32) * inv)
    return jnp.concatenate(outs, axis=1)


def _mega_kernel(gn,
                 xs_ref, xw_ref, ws_ref, bs_ref, ww_ref, bw_ref,
                 wsim_ref, win_ref, wpro_ref, whas_ref,
                 msim_ref, mpro_ref, min_ref, mhas_ref,
                 as_sim_ref, ad_sim_ref, b_sim_ref,
                 as_in_ref, ad_in_ref, b_in_ref,
                 as_pro_ref, ad_pro_ref, b_pro_ref,
                 as_has_ref, ad_has_ref, b_has_ref,
                 w2sim_ref, w2in_ref,
                 as2s_ref, ad2s_ref, b2s_ref,
                 as2i_ref, ad2i_ref, b2i_ref,
                 o_ref,
                 psim, pind, pins, pprod, ppros, phasd, phass,
                 qsim, qind, qins):
    i = pl.program_id(0)

    @pl.when(i < gn)
    def _phase_a():
        rows = pl.ds(i * TILE, TILE)
        hs = jnp.maximum(
            jnp.dot(xs_ref[...].astype(jnp.bfloat16), ws_ref[...],
                    preferred_element_type=jnp.float32) + bs_ref[...], 0.0
        ).astype(jnp.bfloat16)
        hw = jnp.maximum(
            jnp.dot(xw_ref[...].astype(jnp.bfloat16), ww_ref[...],
                    preferred_element_type=jnp.float32) + bw_ref[...], 0.0
        ).astype(jnp.bfloat16)

        def proj(h, w_ref):
            return jnp.dot(h, w_ref[...],
                           preferred_element_type=jnp.float32).astype(jnp.bfloat16)

        psim[rows, :] = proj(hs, wsim_ref)
        pind[rows, :] = proj(hs, win_ref)
        pins[rows, :] = proj(hw, win_ref)
        pprod[rows, :] = proj(hw, wpro_ref)
        ppros[rows, :] = proj(hs, wpro_ref)
        phasd[rows, :] = proj(hw, whas_ref)
        phass[rows, :] = proj(hs, whas_ref)

    @pl.when((i >= gn) & (i < 2 * gn))
    def _phase_b():
        rows = pl.ds((i - gn) * TILE, TILE)
        xd_sim = psim[rows, :]
        h_sent1 = (_gat_relation(xd_sim, psim[...], as_sim_ref, ad_sim_ref,
                                 msim_ref[...]) + b_sim_ref[...]
                   + _gat_relation(pind[rows, :], pins[...], as_in_ref,
                                   ad_in_ref, min_ref[...]) + b_in_ref[...])
        hs1_b = h_sent1.astype(jnp.bfloat16)
        qsim[rows, :] = jnp.dot(hs1_b, w2sim_ref[...],
                                preferred_element_type=jnp.float32).astype(jnp.bfloat16)
        qind[rows, :] = jnp.dot(hs1_b, w2in_ref[...],
                                preferred_element_type=jnp.float32).astype(jnp.bfloat16)

        h_word1 = (_gat_relation(pprod[rows, :], ppros[...], as_pro_ref,
                                 ad_pro_ref, mpro_ref[...]) + b_pro_ref[...]
                   + _gat_relation(phasd[rows, :], phass[...], as_has_ref,
                                   ad_has_ref, mhas_ref[...]) + b_has_ref[...])
        qins[rows, :] = jnp.dot(h_word1.astype(jnp.bfloat16), w2in_ref[...],
                                preferred_element_type=jnp.float32).astype(jnp.bfloat16)

    @pl.when(i >= 2 * gn)
    def _phase_c():
        rows = pl.ds((i - 2 * gn) * TILE, TILE)

        def rel(xd_b, xs_b, as_ref, ad_ref, mask):
            e_src = lax.dot_general(as_ref[...], xs_b, (((1,), (1,)), ((), ())),
                                    preferred_element_type=jnp.float32)
            e_dst = jnp.dot(xd_b, ad_ref[...],
                            preferred_element_type=jnp.float32)
            s = e_dst[:, 0:1] + e_src[0:1, :]
            s = jnp.maximum(s, 0.2 * s)
            p = jnp.exp2(s) * mask
            denom = jnp.sum(p, axis=1, keepdims=True)
            inv = pl.reciprocal(jnp.where(denom > 0, denom, 1.0), approx=True)
            return jnp.dot(p, xs_b, preferred_element_type=jnp.float32) * inv

        o_ref[...] = (rel(qsim[rows, :], qsim[...], as2s_ref, ad2s_ref,
                          msim_ref[...]) + b2s_ref[...]
                      + rel(qind[rows, :], qins[...], as2i_ref, ad2i_ref,
                            min_ref[...]) + b2i_ref[...])


def kernel(sentence_feat, word_feat, mask_similarity, mask_pro_ant, mask_in,
           mask_has, ws, bs, ww, bw,
           conv1_similarity_w, conv1_similarity_as, conv1_similarity_ad, conv1_similarity_b,
           conv2_similarity_w, conv2_similarity_as, conv2_similarity_ad, conv2_similarity_b,
           conv1_pro_ant_w, conv1_pro_ant_as, conv1_pro_ant_ad, conv1_pro_ant_b,
           conv2_pro_ant_w, conv2_pro_ant_as, conv2_pro_ant_ad, conv2_pro_ant_b,
           conv1_in_w, conv1_in_as, conv1_in_ad, conv1_in_b,
           conv2_in_w, conv2_in_as, conv2_in_ad, conv2_in_b,
           conv1_has_w, conv1_has_as, conv1_has_ad, conv1_has_b,
           conv2_has_w, conv2_has_as, conv2_has_ad, conv2_has_b):
    ns, din = sentence_feat.shape
    nw = word_feat.shape[0]
    grid_n = ns // TILE

    sc = jnp.float32(LOG2E)

    def scale(a):
        return (a.astype(jnp.float32) * sc).astype(jnp.bfloat16)

    as_sim, ad_sim = scale(conv1_similarity_as), scale(conv1_similarity_ad)
    as_in, ad_in = scale(conv1_in_as), scale(conv1_in_ad)
    as_pro, ad_pro = scale(conv1_pro_ant_as), scale(conv1_pro_ant_ad)
    as_has, ad_has = scale(conv1_has_as), scale(conv1_has_ad)
    as2_sim, ad2_sim = scale(conv2_similarity_as), scale(conv2_similarity_ad)
    as2_in, ad2_in = scale(conv2_in_as), scale(conv2_in_ad)

    gn = grid_n

    def a_tile(cols):
        return pl.BlockSpec((TILE, cols),
                            lambda i: (jnp.minimum(i, gn - 1), 0))

    def b_tile(cols):
        return pl.BlockSpec((TILE, cols),
                            lambda i: (jnp.clip(i - gn, 0, gn - 1), 0))

    def bc_tile(cols):
        return pl.BlockSpec(
            (TILE, cols),
            lambda i: (jnp.where(i < 2 * gn, jnp.clip(i - gn, 0, gn - 1),
                                 i - 2 * gn), 0))

    out_tile = pl.BlockSpec((TILE, OUT),
                            lambda i: (jnp.clip(i - 2 * gn, 0, gn - 1), 0))

    scratch_p = pltpu.VMEM((ns, HC), jnp.bfloat16)
    scratch_q = pltpu.VMEM((ns, OUT), jnp.bfloat16)

    out = pl.pallas_call(
        functools.partial(_mega_kernel, gn),
        out_shape=jax.ShapeDtypeStruct((ns, OUT), jnp.float32),
        grid=(3 * grid_n,),
        in_specs=[a_tile(din), a_tile(din),
                  _full_spec(ws.shape), _full_spec(bs.shape),
                  _full_spec(ww.shape), _full_spec(bw.shape),
                  _full_spec(conv1_similarity_w.shape),
                  _full_spec(conv1_in_w.shape),
                  _full_spec(conv1_pro_ant_w.shape),
                  _full_spec(conv1_has_w.shape),
                  bc_tile(ns), b_tile(ns), bc_tile(nw), b_tile(ns),
                  _full_spec(conv1_similarity_as.shape),
                  _full_spec(conv1_similarity_ad.shape),
                  _full_spec(conv1_similarity_b.shape),
                  _full_spec(conv1_in_as.shape), _full_spec(conv1_in_ad.shape),
                  _full_spec(conv1_in_b.shape),
                  _full_spec(conv1_pro_ant_as.shape),
                  _full_spec(conv1_pro_ant_ad.shape),
                  _full_spec(conv1_pro_ant_b.shape),
                  _full_spec(conv1_has_as.shape), _full_spec(conv1_has_ad.shape),
                  _full_spec(conv1_has_b.shape),
                  _full_spec(conv2_similarity_w.shape),
                  _full_spec(conv2_in_w.shape),
                  _full_spec(conv2_similarity_as.shape),
                  _full_spec(conv2_similarity_ad.shape),
                  _full_spec(conv2_similarity_b.shape),
                  _full_spec(conv2_in_as.shape), _full_spec(conv2_in_ad.shape),
                  _full_spec(conv2_in_b.shape)],
        out_specs=out_tile,
        scratch_shapes=[scratch_p] * 7 + [scratch_q] * 3,
        compiler_params=pltpu.CompilerParams(
            dimension_semantics=("arbitrary",),
            vmem_limit_bytes=100 * 1024 * 1024),
    )(sentence_feat, word_feat, ws, bs, ww, bw,
      conv1_similarity_w, conv1_in_w, conv1_pro_ant_w, conv1_has_w,
      mask_similarity, mask_pro_ant, mask_in, mask_has,
      as_sim, ad_sim, conv1_similarity_b,
      as_in, ad_in, conv1_in_b,
      as_pro, ad_pro, conv1_pro_ant_b,
      as_has, ad_has, conv1_has_b,
      conv2_similarity_w, conv2_in_w,
      as2_sim, ad2_sim, conv2_similarity_b,
      as2_in, ad2_in, conv2_in_b)

    return out


# speedup vs baseline: 1.3485x; 1.3485x over previous
import functools

import jax
import jax.numpy as jnp
from jax import lax
from jax.experimental import pallas as pl
from jax.experimental.pallas import tpu as pltpu

TILE = 256
HEADS = 8
CH = 64
HC = HEADS * CH
OUT = 128
LOG2E = 1.4426950408889634


def _full_spec(shape):
    return pl.BlockSpec(shape, lambda *_: (0,) * len(shape))


def _gat_relation(xd_b, xs_b, as_ref, ad_ref, mask):
    e_src = lax.dot_general(as_ref[...], xs_b, (((1,), (1,)), ((), ())),
                            preferred_element_type=jnp.float32)
    e_dst = jnp.dot(xd_b, ad_ref[...],
                    preferred_element_type=jnp.float32)
    outs = []
    for h in range(HEADS):
        s = e_dst[:, h:h + 1] + e_src[h:h + 1, :]
        s = jnp.maximum(s, 0.2 * s)
        p = jnp.exp2(s) * mask
        denom = jnp.sum(p, axis=1, keepdims=True)
        inv = pl.reciprocal(jnp.where(denom > 0, denom, 1.0), approx=True)
        outs.append(jnp.dot(p, xs_b[:, h * CH:(h + 1) * CH],
                            preferred_element_type=jnp.float32) * inv)
    return jnp.concatenate(outs, axis=1)


def _mega_kernel(gn,
                 xs_ref, xw_ref, ws_ref, bs_ref, ww_ref, bw_ref,
                 wsim_ref, win_ref, wpro_ref, whas_ref,
                 msim_ref, mpro_ref, min_ref, mhas_ref,
                 as_sim_ref, ad_sim_ref, b_sim_ref,
                 as_in_ref, ad_in_ref, b_in_ref,
                 as_pro_ref, ad_pro_ref, b_pro_ref,
                 as_has_ref, ad_has_ref, b_has_ref,
                 w2sim_ref, w2in_ref,
                 as2s_ref, ad2s_ref, b2s_ref,
                 as2i_ref, ad2i_ref, b2i_ref,
                 o_ref,
                 psim, pind, pins, pprod, ppros, phasd, phass,
                 qsim, qind, qins):
    i = pl.program_id(0)

    @pl.when(i < gn)
    def _phase_a():
        rows = pl.ds(i * TILE, TILE)
        hs = jnp.maximum(
            jnp.dot(xs_ref[...].astype(jnp.bfloat16), ws_ref[...],
                    preferred_element_type=jnp.float32) + bs_ref[...], 0.0
        ).astype(jnp.bfloat16)
        hw = jnp.maximum(
            jnp.dot(xw_ref[...].astype(jnp.bfloat16), ww_ref[...],
                    preferred_element_type=jnp.float32) + bw_ref[...], 0.0
        ).astype(jnp.bfloat16)

        def proj(h, w_ref):
            return jnp.dot(h, w_ref[...],
                           preferred_element_type=jnp.float32).astype(jnp.bfloat16)

        psim[rows, :] = proj(hs, wsim_ref)
        pind[rows, :] = proj(hs, win_ref)
        pins[rows, :] = proj(hw, win_ref)
        pprod[rows, :] = proj(hw, wpro_ref)
        ppros[rows, :] = proj(hs, wpro_ref)
        phasd[rows, :] = proj(hw, whas_ref)
        phass[rows, :] = proj(hs, whas_ref)

    @pl.when((i >= gn) & (i < 2 * gn))
    def _phase_b():
        rows = pl.ds((i - gn) * TILE, TILE)
        xd_sim = psim[rows, :]
        h_sent1 = (_gat_relation(xd_sim, psim[...], as_sim_ref, ad_sim_ref,
                                 msim_ref[...]) + b_sim_ref[...]
                   + _gat_relation(pind[rows, :], pins[...], as_in_ref,
                                   ad_in_ref, min_ref[...]) + b_in_ref[...])
        hs1_b = h_sent1.astype(jnp.bfloat16)
        qsim[rows, :] = jnp.dot(hs1_b, w2sim_ref[...],
                                preferred_element_type=jnp.float32).astype(jnp.bfloat16)
        qind[rows, :] = jnp.dot(hs1_b, w2in_ref[...],
                                preferred_element_type=jnp.float32).astype(jnp.bfloat16)

        h_word1 = (_gat_relation(pprod[rows, :], ppros[...], as_pro_ref,
                                 ad_pro_ref, mpro_ref[...]) + b_pro_ref[...]
                   + _gat_relation(phasd[rows, :], phass[...], as_has_ref,
                                   ad_has_ref, mhas_ref[...]) + b_has_ref[...])
        qins[rows, :] = jnp.dot(h_word1.astype(jnp.bfloat16), w2in_ref[...],
                                preferred_element_type=jnp.float32).astype(jnp.bfloat16)

    @pl.when(i >= 2 * gn)
    def _phase_c():
        rows = pl.ds((i - 2 * gn) * TILE, TILE)

        def rel(xd_b, xs_b, as_ref, ad_ref, mask):
            e_src = lax.dot_general(as_ref[...], xs_b, (((1,), (1,)), ((), ())),
                                    preferred_element_type=jnp.float32)
            e_dst = jnp.dot(xd_b, ad_ref[...],
                            preferred_element_type=jnp.float32)
            s = e_dst[:, 0:1] + e_src[0:1, :]
            s = jnp.maximum(s, 0.2 * s)
            p = jnp.exp2(s) * mask
            denom = jnp.sum(p, axis=1, keepdims=True)
            inv = pl.reciprocal(jnp.where(denom > 0, denom, 1.0), approx=True)
            return jnp.dot(p, xs_b, preferred_element_type=jnp.float32) * inv

        o_ref[...] = (rel(qsim[rows, :], qsim[...], as2s_ref, ad2s_ref,
                          msim_ref[...]) + b2s_ref[...]
                      + rel(qind[rows, :], qins[...], as2i_ref, ad2i_ref,
                            min_ref[...]) + b2i_ref[...])


def kernel(sentence_feat, word_feat, mask_similarity, mask_pro_ant, mask_in,
           mask_has, ws, bs, ww, bw,
           conv1_similarity_w, conv1_similarity_as, conv1_similarity_ad, conv1_similarity_b,
           conv2_similarity_w, conv2_similarity_as, conv2_similarity_ad, conv2_similarity_b,
           conv1_pro_ant_w, conv1_pro_ant_as, conv1_pro_ant_ad, conv1_pro_ant_b,
           conv2_pro_ant_w, conv2_pro_ant_as, conv2_pro_ant_ad, conv2_pro_ant_b,
           conv1_in_w, conv1_in_as, conv1_in_ad, conv1_in_b,
           conv2_in_w, conv2_in_as, conv2_in_ad, conv2_in_b,
           conv1_has_w, conv1_has_as, conv1_has_ad, conv1_has_b,
           conv2_has_w, conv2_has_as, conv2_has_ad, conv2_has_b):
    ns, din = sentence_feat.shape
    nw = word_feat.shape[0]
    grid_n = ns // TILE

    sc = jnp.float32(LOG2E)

    def scale(a):
        return (a.astype(jnp.float32) * sc).astype(jnp.bfloat16)

    as_sim, ad_sim = scale(conv1_similarity_as), scale(conv1_similarity_ad)
    as_in, ad_in = scale(conv1_in_as), scale(conv1_in_ad)
    as_pro, ad_pro = scale(conv1_pro_ant_as), scale(conv1_pro_ant_ad)
    as_has, ad_has = scale(conv1_has_as), scale(conv1_has_ad)
    as2_sim, ad2_sim = scale(conv2_similarity_as), scale(conv2_similarity_ad)
    as2_in, ad2_in = scale(conv2_in_as), scale(conv2_in_ad)

    gn = grid_n

    def a_tile(cols):
        return pl.BlockSpec((TILE, cols),
                            lambda i: (jnp.minimum(i, gn - 1), 0))

    def b_tile(cols):
        return pl.BlockSpec((TILE, cols),
                            lambda i: (jnp.clip(i - gn, 0, gn - 1), 0))

    def bc_tile(cols):
        return pl.BlockSpec(
            (TILE, cols),
            lambda i: (jnp.where(i < 2 * gn, jnp.clip(i - gn, 0, gn - 1),
                                 i - 2 * gn), 0))

    out_tile = pl.BlockSpec((TILE, OUT),
                            lambda i: (jnp.clip(i - 2 * gn, 0, gn - 1), 0))

    scratch_p = pltpu.VMEM((ns, HC), jnp.bfloat16)
    scratch_q = pltpu.VMEM((ns, OUT), jnp.bfloat16)

    out = pl.pallas_call(
        functools.partial(_mega_kernel, gn),
        out_shape=jax.ShapeDtypeStruct((ns, OUT), jnp.float32),
        grid=(3 * grid_n,),
        in_specs=[a_tile(din), a_tile(din),
                  _full_spec(ws.shape), _full_spec(bs.shape),
                  _full_spec(ww.shape), _full_spec(bw.shape),
                  _full_spec(conv1_similarity_w.shape),
                  _full_spec(conv1_in_w.shape),
                  _full_spec(conv1_pro_ant_w.shape),
                  _full_spec(conv1_has_w.shape),
                  bc_tile(ns), b_tile(ns), bc_tile(nw), b_tile(ns),
                  _full_spec(conv1_similarity_as.shape),
                  _full_spec(conv1_similarity_ad.shape),
                  _full_spec(conv1_similarity_b.shape),
                  _full_spec(conv1_in_as.shape), _full_spec(conv1_in_ad.shape),
                  _full_spec(conv1_in_b.shape),
                  _full_spec(conv1_pro_ant_as.shape),
                  _full_spec(conv1_pro_ant_ad.shape),
                  _full_spec(conv1_pro_ant_b.shape),
                  _full_spec(conv1_has_as.shape), _full_spec(conv1_has_ad.shape),
                  _full_spec(conv1_has_b.shape),
                  _full_spec(conv2_similarity_w.shape),
                  _full_spec(conv2_in_w.shape),
                  _full_spec(conv2_similarity_as.shape),
                  _full_spec(conv2_similarity_ad.shape),
                  _full_spec(conv2_similarity_b.shape),
                  _full_spec(conv2_in_as.shape), _full_spec(conv2_in_ad.shape),
                  _full_spec(conv2_in_b.shape)],
        out_specs=out_tile,
        scratch_shapes=[scratch_p] * 7 + [scratch_q] * 3,
        compiler_params=pltpu.CompilerParams(
            dimension_semantics=("arbitrary",),
            vmem_limit_bytes=100 * 1024 * 1024),
    )(sentence_feat, word_feat, ws, bs, ww, bw,
      conv1_similarity_w, conv1_in_w, conv1_pro_ant_w, conv1_has_w,
      mask_similarity, mask_pro_ant, mask_in, mask_has,
      as_sim, ad_sim, conv1_similarity_b,
      as_in, ad_in, conv1_in_b,
      as_pro, ad_pro, conv1_pro_ant_b,
      as_has, ad_has, conv1_has_b,
      conv2_similarity_w, conv2_in_w,
      as2_sim, ad2_sim, conv2_similarity_b,
      as2_in, ad2_in, conv2_in_b)

    return out
